# Initial kernel scaffold; baseline (speedup 1.0000x reference)
import jax
import jax.numpy as jnp
from jax.experimental import pallas as pl


def kernel(x, embeddings):
    raise NotImplementedError("write your pallas kernel here")


# speedup vs baseline: 3.2694x; 3.2694x over previous
import jax
import jax.numpy as jnp
from jax.experimental import pallas as pl

_CHUNK = 256


def _add_pos_kernel(x_ref, emb_ref, out_ref):
    out_ref[...] = x_ref[...] + emb_ref[...][None, :, :]


def kernel(x, embeddings):
    B, S, D = x.shape
    grid = (S // _CHUNK,)
    return pl.pallas_call(
        _add_pos_kernel,
        grid=grid,
        in_specs=[
            pl.BlockSpec((B, _CHUNK, D), lambda i: (0, i, 0)),
            pl.BlockSpec((_CHUNK, D), lambda i: (i, 0)),
        ],
        out_specs=pl.BlockSpec((B, _CHUNK, D), lambda i: (0, i, 0)),
        out_shape=jax.ShapeDtypeStruct((B, S, D), x.dtype),
    )(x, embeddings)
